# baseline (speedup 1.0000x reference)
import functools

import jax
import jax.numpy as jnp
from jax import lax
from jax.experimental import pallas as pl
from jax.experimental.pallas import tpu as pltpu
from jax.experimental.pallas import tpu_sc as plsc

B = 16
N = 4096
D = 64
TEMPERATURE = 0.05

NW = 32
CHUNK = 128
NCHUNK = N // CHUNK
GROUPS = CHUNK // 16


def _bce_weighted(dots, t, wv):
    l = dots * (1.0 / TEMPERATURE)
    al = jnp.abs(l)
    u = jnp.exp(-al)
    z = u / (u + 2.0)
    z2 = z * z
    lp = (2.0 * z) * (1.0 + z2 * (1.0 / 3 + z2 * (1.0 / 5 + z2 * (1.0 / 7 + z2 * (1.0 / 9)))))
    loss = jnp.maximum(l, 0.0) - l * t + lp
    return loss * wv


SPLIT = 8


def _sc_loss_parts(lengths, output_embeddings, supervision_embeddings,
                   supervision_weights, supervision_ratings):
    mesh = plsc.VectorSubcoreMesh(core_axis_name="c", subcore_axis_name="s")

    @functools.partial(
        pl.kernel,
        mesh=mesh,
        compiler_params=pltpu.CompilerParams(needs_layout_passes=False),
        out_type=[
            jax.ShapeDtypeStruct((NW, 16), jnp.float32),
            jax.ShapeDtypeStruct((NW, 16), jnp.float32),
        ],
        scratch_types=[
            pltpu.VMEM((B,), jnp.int32),
            pltpu.VMEM((CHUNK, D), jnp.float32),
            pltpu.VMEM((CHUNK, D), jnp.float32),
            pltpu.VMEM((CHUNK, D), jnp.float32),
            pltpu.VMEM((CHUNK, D), jnp.float32),
            pltpu.VMEM((B * CHUNK,), jnp.float32),
            pltpu.VMEM((B * CHUNK,), jnp.float32),
            pltpu.VMEM((16,), jnp.float32),
            pltpu.VMEM((16,), jnp.float32),
            pltpu.SemaphoreType.DMA,
            pltpu.SemaphoreType.DMA,
            pltpu.SemaphoreType.DMA,
        ],
    )
    def sc_kernel(len_hbm, oe_hbm, se_hbm, sw_hbm, sr_hbm,
                  wl_out, w_out,
                  len_v, a0_v, a1_v, c0_v, c1_v, w_v, r_v, awl_v, aw_v,
                  sem0, sem1, semwr):
        wid = lax.axis_index("s") * 2 + lax.axis_index("c")
        lane = lax.iota(jnp.int32, 16)

        for b in range(SPLIT):
            base = ((wid + 2 * b) & (NCHUNK - 1)) * CHUNK
            pltpu.async_copy(sw_hbm.at[b, pl.ds(base, CHUNK)],
                             w_v.at[pl.ds(b * CHUNK, CHUNK)], semwr)
            pltpu.async_copy(sr_hbm.at[b, pl.ds(base, CHUNK)],
                             r_v.at[pl.ds(b * CHUNK, CHUNK)], semwr)
        pltpu.sync_copy(len_hbm, len_v)
        for b in range(SPLIT):
            base = ((wid + 2 * b) & (NCHUNK - 1)) * CHUNK
            pltpu.make_async_copy(sw_hbm.at[b, pl.ds(base, CHUNK)],
                                  w_v.at[pl.ds(b * CHUNK, CHUNK)], semwr).wait()
            pltpu.make_async_copy(sr_hbm.at[b, pl.ds(base, CHUNK)],
                                  r_v.at[pl.ds(b * CHUNK, CHUNK)], semwr).wait()

        awl_v[...] = jnp.zeros((16,), jnp.float32)
        aw_v[...] = jnp.zeros((16,), jnp.float32)
        len_all = len_v[...]

        def binfo(b):
            base = ((wid + 2 * b) & (NCHUNK - 1)) * CHUNK
            len_b = jnp.max(jnp.where(lane == b, len_all, 0))
            return b * N + base, base, len_b

        def issue(b, a_buf, c_buf, sem):
            grow, base, len_b = binfo(b)

            @pl.when(base < len_b)
            def _():
                pltpu.async_copy(oe_hbm.at[pl.ds(grow, CHUNK)], a_buf, sem)
                pltpu.async_copy(se_hbm.at[pl.ds(grow, CHUNK)], c_buf, sem)

        def compute(b, a_buf, c_buf, sem):
            grow, base, len_b = binfo(b)

            @pl.when(base < len_b)
            def _():
                pltpu.make_async_copy(oe_hbm.at[pl.ds(grow, CHUNK)], a_buf, sem).wait()
                pltpu.make_async_copy(se_hbm.at[pl.ds(grow, CHUNK)], c_buf, sem).wait()

                def group_body(g, _):
                    row0 = g * 16
                    rows = row0 + lane
                    accs = [jnp.zeros((16,), jnp.float32) for _ in range(4)]
                    for d in range(D):
                        dvec = (lane + d) & (D - 1)
                        a = plsc.load_gather(a_buf, [rows, dvec])
                        c = plsc.load_gather(c_buf, [rows, dvec])
                        accs[d & 3] = accs[d & 3] + a * c
                    dots = (accs[0] + accs[1]) + (accs[2] + accs[3])
                    off = b * CHUNK + row0
                    t = r_v[pl.ds(off, 16)]
                    wv = w_v[pl.ds(off, 16)]
                    valid = (base + rows) < len_b
                    wv = jnp.where(valid, wv, 0.0)
                    awl_v[...] += _bce_weighted(dots, t, wv)
                    aw_v[...] += wv
                    return _

                lax.fori_loop(0, GROUPS, group_body, None)

        issue(0, a0_v, c0_v, sem0)

        def pipe_body(i, _):
            b0 = 2 * i
            issue(b0 + 1, a1_v, c1_v, sem1)
            compute(b0, a0_v, c0_v, sem0)

            @pl.when(i < SPLIT // 2 - 1)
            def _():
                issue(b0 + 2, a0_v, c0_v, sem0)

            compute(b0 + 1, a1_v, c1_v, sem1)
            return _

        lax.fori_loop(0, SPLIT // 2, pipe_body, None)

        pltpu.sync_copy(awl_v, wl_out.at[wid])
        pltpu.sync_copy(aw_v, w_out.at[wid])

    return sc_kernel(lengths,
                     output_embeddings.reshape(B * N, D),
                     supervision_embeddings.reshape(B * N, D),
                     supervision_weights, supervision_ratings)


BLK = 1024
NC = N // BLK
G = BLK // 128
BSH = BLK.bit_length() - 1


def _tc_body(len_ref, oe_ref, se_ref, w_ref, r_ref, wl_ref, w_out_ref):
    b = pl.program_id(0)
    j = pl.program_id(1)
    len_b = len_ref[b]

    @pl.when(j == 0)
    def _():
        wl_ref[...] = jnp.zeros_like(wl_ref)
        w_out_ref[...] = jnp.zeros_like(w_out_ref)

    @pl.when(j * BLK < len_b)
    def _():
        p = oe_ref[0] * se_ref[0]
        ones = jnp.ones((1, D), jnp.float32)
        dots = jnp.concatenate(
            [lax.dot_general(ones, p[g * 128:(g + 1) * 128, :],
                             (((1,), (1,)), ((), ())),
                             preferred_element_type=jnp.float32)
             for g in range(G)], axis=0)
        l = dots * (1.0 / TEMPERATURE)
        t = r_ref[...]
        bce = jnp.maximum(l, 0.0) - l * t + jnp.log1p(jnp.exp(-jnp.abs(l)))
        rows = (j * BLK
                + lax.broadcasted_iota(jnp.int32, (G, 128), 0) * 128
                + lax.broadcasted_iota(jnp.int32, (G, 128), 1))
        wv = jnp.where(rows < len_b, w_ref[...], 0.0)
        wl_ref[0] += wv * bce
        w_out_ref[0] += wv


def _tc_loss_parts(lengths, oe, se, w, r):
    nb = oe.shape[0]

    def emb_map(b, j, len_ref):
        return (b, jnp.minimum(j, (len_ref[b] - 1) >> BSH), 0)

    def vec_map(b, j, len_ref):
        return (b * NC + jnp.minimum(j, (len_ref[b] - 1) >> BSH), 0)

    def out_map(b, j, len_ref):
        return (b, 0, 0)

    grid_spec = pltpu.PrefetchScalarGridSpec(
        num_scalar_prefetch=1,
        grid=(nb, NC),
        in_specs=[
            pl.BlockSpec((1, BLK, D), emb_map),
            pl.BlockSpec((1, BLK, D), emb_map),
            pl.BlockSpec((G, 128), vec_map),
            pl.BlockSpec((G, 128), vec_map),
        ],
        out_specs=[
            pl.BlockSpec((1, G, 128), out_map),
            pl.BlockSpec((1, G, 128), out_map),
        ],
    )
    return pl.pallas_call(
        _tc_body,
        grid_spec=grid_spec,
        out_shape=[
            jax.ShapeDtypeStruct((nb, G, 128), jnp.float32),
            jax.ShapeDtypeStruct((nb, G, 128), jnp.float32),
        ],
    )(lengths, oe, se,
      w.reshape(nb * N // 128, 128), r.reshape(nb * N // 128, 128))


def kernel(lengths, output_embeddings, supervision_ids, supervision_embeddings,
           supervision_weights, supervision_ratings):
    del supervision_ids
    sc_wl, sc_w = _sc_loss_parts(lengths, output_embeddings,
                                 supervision_embeddings, supervision_weights,
                                 supervision_ratings)
    tc_wl, tc_w = _tc_loss_parts(lengths[SPLIT:], output_embeddings[SPLIT:],
                                 supervision_embeddings[SPLIT:],
                                 supervision_weights[SPLIT:],
                                 supervision_ratings[SPLIT:])
    num = jnp.sum(sc_wl) + jnp.sum(tc_wl)
    den = jnp.sum(sc_w) + jnp.sum(tc_w)
    return num / den


# speedup vs baseline: 1.3055x; 1.3055x over previous
import functools

import jax
import jax.numpy as jnp
from jax import lax
from jax.experimental import pallas as pl
from jax.experimental.pallas import tpu as pltpu
from jax.experimental.pallas import tpu_sc as plsc

B = 16
N = 4096
D = 64
TEMPERATURE = 0.05

NW = 32
CHUNK = 128
NCHUNK = N // CHUNK
GROUPS = CHUNK // 16


def _bce_weighted(dots, t, wv):
    l = dots * (1.0 / TEMPERATURE)
    al = jnp.abs(l)
    u = jnp.exp(-al)
    z = u / (u + 2.0)
    z2 = z * z
    lp = (2.0 * z) * (1.0 + z2 * (1.0 / 3 + z2 * (1.0 / 5 + z2 * (1.0 / 7 + z2 * (1.0 / 9)))))
    loss = jnp.maximum(l, 0.0) - l * t + lp
    return loss * wv


def _sc_loss_parts(lengths, output_embeddings, supervision_embeddings,
                   supervision_weights, supervision_ratings):
    mesh = plsc.VectorSubcoreMesh(core_axis_name="c", subcore_axis_name="s")

    @functools.partial(
        pl.kernel,
        mesh=mesh,
        compiler_params=pltpu.CompilerParams(needs_layout_passes=False),
        out_type=[
            jax.ShapeDtypeStruct((NW, 16), jnp.float32),
            jax.ShapeDtypeStruct((NW, 16), jnp.float32),
        ],
        scratch_types=[
            pltpu.VMEM((B,), jnp.int32),
            pltpu.VMEM((CHUNK, D), jnp.float32),
            pltpu.VMEM((CHUNK, D), jnp.float32),
            pltpu.VMEM((CHUNK, D), jnp.float32),
            pltpu.VMEM((CHUNK, D), jnp.float32),
            pltpu.VMEM((B * CHUNK,), jnp.float32),
            pltpu.VMEM((B * CHUNK,), jnp.float32),
            pltpu.VMEM((16,), jnp.float32),
            pltpu.VMEM((16,), jnp.float32),
            pltpu.SemaphoreType.DMA,
            pltpu.SemaphoreType.DMA,
            pltpu.SemaphoreType.DMA,
        ],
    )
    def sc_kernel(len_hbm, oe_hbm, se_hbm, sw_hbm, sr_hbm,
                  wl_out, w_out,
                  len_v, a0_v, a1_v, c0_v, c1_v, w_v, r_v, awl_v, aw_v,
                  sem0, sem1, semwr):
        wid = lax.axis_index("s") * 2 + lax.axis_index("c")
        lane = lax.iota(jnp.int32, 16)

        for b in range(B):
            base = ((wid + 2 * b) & (NCHUNK - 1)) * CHUNK
            pltpu.async_copy(sw_hbm.at[b, pl.ds(base, CHUNK)],
                             w_v.at[pl.ds(b * CHUNK, CHUNK)], semwr)
            pltpu.async_copy(sr_hbm.at[b, pl.ds(base, CHUNK)],
                             r_v.at[pl.ds(b * CHUNK, CHUNK)], semwr)
        pltpu.sync_copy(len_hbm, len_v)
        for b in range(B):
            base = ((wid + 2 * b) & (NCHUNK - 1)) * CHUNK
            pltpu.make_async_copy(sw_hbm.at[b, pl.ds(base, CHUNK)],
                                  w_v.at[pl.ds(b * CHUNK, CHUNK)], semwr).wait()
            pltpu.make_async_copy(sr_hbm.at[b, pl.ds(base, CHUNK)],
                                  r_v.at[pl.ds(b * CHUNK, CHUNK)], semwr).wait()

        awl_v[...] = jnp.zeros((16,), jnp.float32)
        aw_v[...] = jnp.zeros((16,), jnp.float32)
        len_all = len_v[...]

        def binfo(b):
            base = ((wid + 2 * b) & (NCHUNK - 1)) * CHUNK
            len_b = jnp.max(jnp.where(lane == b, len_all, 0))
            return b * N + base, base, len_b

        def issue(b, a_buf, c_buf, sem):
            grow, base, len_b = binfo(b)

            @pl.when(base < len_b)
            def _():
                pltpu.async_copy(oe_hbm.at[pl.ds(grow, CHUNK)], a_buf, sem)
                pltpu.async_copy(se_hbm.at[pl.ds(grow, CHUNK)], c_buf, sem)

        def compute(b, a_buf, c_buf, sem):
            grow, base, len_b = binfo(b)

            @pl.when(base < len_b)
            def _():
                pltpu.make_async_copy(oe_hbm.at[pl.ds(grow, CHUNK)], a_buf, sem).wait()
                pltpu.make_async_copy(se_hbm.at[pl.ds(grow, CHUNK)], c_buf, sem).wait()

                def group_body(g, _):
                    row0 = g * 16
                    rows = row0 + lane
                    accs = [jnp.zeros((16,), jnp.float32) for _ in range(4)]
                    for d in range(D):
                        dvec = (lane + d) & (D - 1)
                        a = plsc.load_gather(a_buf, [rows, dvec])
                        c = plsc.load_gather(c_buf, [rows, dvec])
                        accs[d & 3] = accs[d & 3] + a * c
                    dots = (accs[0] + accs[1]) + (accs[2] + accs[3])
                    off = b * CHUNK + row0
                    t = r_v[pl.ds(off, 16)]
                    wv = w_v[pl.ds(off, 16)]
                    valid = (base + rows) < len_b
                    wv = jnp.where(valid, wv, 0.0)
                    awl_v[...] += _bce_weighted(dots, t, wv)
                    aw_v[...] += wv
                    return _

                lax.fori_loop(0, GROUPS, group_body, None)

        issue(0, a0_v, c0_v, sem0)

        def pipe_body(i, _):
            b0 = 2 * i
            issue(b0 + 1, a1_v, c1_v, sem1)
            compute(b0, a0_v, c0_v, sem0)

            @pl.when(i < B // 2 - 1)
            def _():
                issue(b0 + 2, a0_v, c0_v, sem0)

            compute(b0 + 1, a1_v, c1_v, sem1)
            return _

        lax.fori_loop(0, B // 2, pipe_body, None)

        pltpu.sync_copy(awl_v, wl_out.at[wid])
        pltpu.sync_copy(aw_v, w_out.at[wid])

    return sc_kernel(lengths,
                     output_embeddings.reshape(B * N, D),
                     supervision_embeddings.reshape(B * N, D),
                     supervision_weights, supervision_ratings)


def kernel(lengths, output_embeddings, supervision_ids, supervision_embeddings,
           supervision_weights, supervision_ratings):
    del supervision_ids
    wl, w = _sc_loss_parts(lengths, output_embeddings, supervision_embeddings,
                           supervision_weights, supervision_ratings)
    return jnp.sum(wl) / jnp.sum(w)
